# baseline (speedup 1.0000x reference)
import jax
import jax.numpy as jnp
from jax.experimental import pallas as pl
from jax.experimental.pallas import tpu as pltpu

_TM = 256


def _fused_kernel(H_ref, W_ref, b_ref, gamma_ref, beta_ref, A_ref,
                  out_ref, lin_ref):
    @pl.when(pl.program_id(0) == 0)
    def _prologue():
        h = H_ref[...]
        nrm = jnp.sqrt(jnp.sum(h * h, axis=1, keepdims=True))
        hn = h / jnp.maximum(nrm, 1e-12)
        mu = jnp.mean(hn, axis=0, keepdims=True)
        xc = hn - mu
        var = jnp.mean(xc * xc, axis=0, keepdims=True)
        hbn = xc * jax.lax.rsqrt(var + 1e-5) * gamma_ref[...] + beta_ref[...]
        lin = jax.lax.dot_general(
            hbn, W_ref[...], (((1,), (1,)), ((), ())),
            preferred_element_type=jnp.float32) + b_ref[...]
        lin_ref[...] = lin

    acc = jnp.dot(A_ref[...], lin_ref[...],
                  preferred_element_type=jnp.float32,
                  precision=jax.lax.Precision.DEFAULT)
    out_ref[...] = jnp.where(acc >= 0, acc, 0.01 * acc)


def kernel(H, A_norm, W, b, gamma, beta):
    n, d_in = H.shape
    d_out = W.shape[0]
    return pl.pallas_call(
        _fused_kernel,
        grid=(n // _TM,),
        in_specs=[
            pl.BlockSpec((n, d_in), lambda i: (0, 0)),
            pl.BlockSpec((d_out, d_in), lambda i: (0, 0)),
            pl.BlockSpec((1, d_out), lambda i: (0, 0)),
            pl.BlockSpec((1, d_in), lambda i: (0, 0)),
            pl.BlockSpec((1, d_in), lambda i: (0, 0)),
            pl.BlockSpec((_TM, n), lambda i: (i, 0)),
        ],
        out_specs=pl.BlockSpec((_TM, d_out), lambda i: (i, 0)),
        out_shape=jax.ShapeDtypeStruct((n, d_out), jnp.float32),
        scratch_shapes=[pltpu.VMEM((n, d_out), jnp.float32)],
        compiler_params=pltpu.CompilerParams(
            dimension_semantics=("arbitrary",)),
    )(H, W, b.reshape(1, d_out), gamma.reshape(1, d_in),
      beta.reshape(1, d_in), A_norm)


# speedup vs baseline: 1.0212x; 1.0212x over previous
import jax
import jax.numpy as jnp
from jax.experimental import pallas as pl
from jax.experimental.pallas import tpu as pltpu

_TM = 1024


def _fused_kernel(H_ref, W_ref, b_ref, gamma_ref, beta_ref,
                  A0_ref, A1_ref, A2_ref, A3_ref, out_ref, lin_ref):
    @pl.when(pl.program_id(0) == 0)
    def _prologue():
        h = H_ref[...]
        nrm = jnp.sqrt(jnp.sum(h * h, axis=1, keepdims=True))
        hn = h / jnp.maximum(nrm, 1e-12)
        mu = jnp.mean(hn, axis=0, keepdims=True)
        xc = hn - mu
        var = jnp.mean(xc * xc, axis=0, keepdims=True)
        hbn = xc * jax.lax.rsqrt(var + 1e-5) * gamma_ref[...] + beta_ref[...]
        lin = jax.lax.dot_general(
            hbn, W_ref[...], (((1,), (1,)), ((), ())),
            preferred_element_type=jnp.float32) + b_ref[...]
        lin_ref[...] = lin

    nq = lin_ref.shape[0] // 4
    acc = jnp.dot(A0_ref[...], lin_ref[:nq, :],
                  preferred_element_type=jnp.float32,
                  precision=jax.lax.Precision.DEFAULT)
    acc += jnp.dot(A1_ref[...], lin_ref[nq:2 * nq, :],
                   preferred_element_type=jnp.float32,
                   precision=jax.lax.Precision.DEFAULT)
    acc += jnp.dot(A2_ref[...], lin_ref[2 * nq:3 * nq, :],
                   preferred_element_type=jnp.float32,
                   precision=jax.lax.Precision.DEFAULT)
    acc += jnp.dot(A3_ref[...], lin_ref[3 * nq:, :],
                   preferred_element_type=jnp.float32,
                   precision=jax.lax.Precision.DEFAULT)
    out_ref[...] = jnp.where(acc >= 0, acc, 0.01 * acc)


def kernel(H, A_norm, W, b, gamma, beta):
    n, d_in = H.shape
    d_out = W.shape[0]
    nq = n // 4
    aspec = lambda q: pl.BlockSpec((_TM, nq), lambda i, q=q: (i, q))
    return pl.pallas_call(
        _fused_kernel,
        grid=(n // _TM,),
        in_specs=[
            pl.BlockSpec((n, d_in), lambda i: (0, 0)),
            pl.BlockSpec((d_out, d_in), lambda i: (0, 0)),
            pl.BlockSpec((1, d_out), lambda i: (0, 0)),
            pl.BlockSpec((1, d_in), lambda i: (0, 0)),
            pl.BlockSpec((1, d_in), lambda i: (0, 0)),
            aspec(0), aspec(1), aspec(2), aspec(3),
        ],
        out_specs=pl.BlockSpec((_TM, d_out), lambda i: (i, 0)),
        out_shape=jax.ShapeDtypeStruct((n, d_out), jnp.float32),
        scratch_shapes=[pltpu.VMEM((n, d_out), jnp.float32)],
        compiler_params=pltpu.CompilerParams(
            dimension_semantics=("arbitrary",)),
    )(H, W, b.reshape(1, d_out), gamma.reshape(1, d_in),
      beta.reshape(1, d_in), A_norm, A_norm, A_norm, A_norm)


# speedup vs baseline: 1.0572x; 1.0352x over previous
import jax
import jax.numpy as jnp
from jax.experimental import pallas as pl
from jax.experimental.pallas import tpu as pltpu

_TM = 512
_CHUNKS = 8
_SLOTS = 3


def _fused_kernel(H_ref, W_ref, b_ref, gamma_ref, beta_ref, A_hbm,
                  out_ref, lin_ref, abuf_ref, sems):
    n = out_ref.shape[0]
    ntiles = n // _TM
    rows = _TM // _CHUNKS

    def chunk_copy(t, c):
        r0 = t * _TM + c * rows
        return pltpu.make_async_copy(
            A_hbm.at[pl.ds(r0, rows), :],
            abuf_ref.at[t % _SLOTS, pl.ds(c * rows, rows), :],
            sems.at[t % _SLOTS])

    def start_tile(t):
        for c in range(_CHUNKS):
            chunk_copy(t, c).start()

    def wait_tile(t):
        for c in range(_CHUNKS):
            chunk_copy(t, c).wait()

    for t in range(min(_SLOTS, ntiles)):
        start_tile(t)

    h = H_ref[...]
    nrm = jnp.sqrt(jnp.sum(h * h, axis=1, keepdims=True))
    hn = h / jnp.maximum(nrm, 1e-12)
    mu = jnp.mean(hn, axis=0, keepdims=True)
    xc = hn - mu
    var = jnp.mean(xc * xc, axis=0, keepdims=True)
    hbn = xc * jax.lax.rsqrt(var + 1e-5) * gamma_ref[...] + beta_ref[...]
    lin = jax.lax.dot_general(
        hbn, W_ref[...], (((1,), (1,)), ((), ())),
        preferred_element_type=jnp.float32) + b_ref[...]
    lin_ref[...] = lin

    for t in range(ntiles):
        wait_tile(t)
        acc = jnp.dot(abuf_ref[t % _SLOTS], lin_ref[...],
                      preferred_element_type=jnp.float32,
                      precision=jax.lax.Precision.DEFAULT)
        out_ref[t * _TM:(t + 1) * _TM, :] = jnp.where(acc >= 0, acc, 0.01 * acc)
        if t + _SLOTS < ntiles:
            start_tile(t + _SLOTS)


def kernel(H, A_norm, W, b, gamma, beta):
    n, d_in = H.shape
    d_out = W.shape[0]
    return pl.pallas_call(
        _fused_kernel,
        in_specs=[
            pl.BlockSpec(memory_space=pltpu.MemorySpace.VMEM),
            pl.BlockSpec(memory_space=pltpu.MemorySpace.VMEM),
            pl.BlockSpec(memory_space=pltpu.MemorySpace.VMEM),
            pl.BlockSpec(memory_space=pltpu.MemorySpace.VMEM),
            pl.BlockSpec(memory_space=pltpu.MemorySpace.VMEM),
            pl.BlockSpec(memory_space=pltpu.MemorySpace.HBM),
        ],
        out_specs=pl.BlockSpec(memory_space=pltpu.MemorySpace.VMEM),
        out_shape=jax.ShapeDtypeStruct((n, d_out), jnp.float32),
        scratch_shapes=[
            pltpu.VMEM((n, d_out), jnp.float32),
            pltpu.VMEM((_SLOTS, _TM, n), jnp.float32),
            pltpu.SemaphoreType.DMA((_SLOTS,)),
        ],
    )(H, W, b.reshape(1, d_out), gamma.reshape(1, d_in),
      beta.reshape(1, d_in), A_norm)


# speedup vs baseline: 1.1225x; 1.0618x over previous
import jax
import jax.numpy as jnp
from jax.experimental import pallas as pl
from jax.experimental.pallas import tpu as pltpu

_TM = 1024


def _fused_kernel(H_ref, W_ref, b_ref, gamma_ref, beta_ref, A_ref,
                  out_ref, lin_ref):
    @pl.when(pl.program_id(0) == 0)
    def _prologue():
        h = H_ref[...]
        nrm = jnp.sqrt(jnp.sum(h * h, axis=1, keepdims=True))
        hn = h / jnp.maximum(nrm, 1e-12)
        mu = jnp.mean(hn, axis=0, keepdims=True)
        xc = hn - mu
        var = jnp.mean(xc * xc, axis=0, keepdims=True)
        hbn = xc * jax.lax.rsqrt(var + 1e-5) * gamma_ref[...] + beta_ref[...]
        lin = jax.lax.dot_general(
            hbn, W_ref[...], (((1,), (1,)), ((), ())),
            preferred_element_type=jnp.float32) + b_ref[...]
        lin_ref[...] = lin

    acc = jnp.dot(A_ref[...], lin_ref[...],
                  preferred_element_type=jnp.float32,
                  precision=jax.lax.Precision.DEFAULT)
    out_ref[...] = jnp.where(acc >= 0, acc, 0.01 * acc)


def kernel(H, A_norm, W, b, gamma, beta):
    n, d_in = H.shape
    d_out = W.shape[0]
    return pl.pallas_call(
        _fused_kernel,
        grid=(n // _TM,),
        in_specs=[
            pl.BlockSpec((n, d_in), lambda i: (0, 0)),
            pl.BlockSpec((d_out, d_in), lambda i: (0, 0)),
            pl.BlockSpec((1, d_out), lambda i: (0, 0)),
            pl.BlockSpec((1, d_in), lambda i: (0, 0)),
            pl.BlockSpec((1, d_in), lambda i: (0, 0)),
            pl.BlockSpec((_TM, n), lambda i: (i, 0)),
        ],
        out_specs=pl.BlockSpec((_TM, d_out), lambda i: (i, 0)),
        out_shape=jax.ShapeDtypeStruct((n, d_out), jnp.float32),
        scratch_shapes=[pltpu.VMEM((n, d_out), jnp.float32)],
        compiler_params=pltpu.CompilerParams(
            dimension_semantics=("arbitrary",)),
    )(H, W, b.reshape(1, d_out), gamma.reshape(1, d_in),
      beta.reshape(1, d_in), A_norm)
